# baseline (speedup 1.0000x reference)
import functools

import jax
from jax import lax
import jax.numpy as jnp
from jax.experimental import pallas as pl
from jax.experimental.pallas import tpu as pltpu
from jax.experimental.pallas import tpu_sc as plsc

_NW = 32


def _sc_row_gather(table, idx, chunk):
    V, D = table.shape
    B = idx.shape[0]
    b_per_w = B // _NW
    nch = b_per_w // chunk
    idx3 = idx.reshape(_NW, nch, chunk)
    mesh = plsc.VectorSubcoreMesh(core_axis_name="c", subcore_axis_name="s")

    @functools.partial(
        pl.kernel,
        mesh=mesh,
        out_type=jax.ShapeDtypeStruct((B, D), table.dtype),
        scratch_types=[
            pltpu.VMEM((nch, chunk), jnp.int32),
            pltpu.VMEM((2, chunk, D), table.dtype),
            pltpu.SemaphoreType.DMA,
        ],
    )
    def k(table_hbm, idx_hbm, out_hbm, idx_v, buf_v, sem):
        wid = lax.axis_index("s") * 2 + lax.axis_index("c")
        base = wid * b_per_w
        pltpu.sync_copy(idx_hbm.at[wid], idx_v)
        cps = [pltpu.make_async_copy(table_hbm.at[idx_v.at[0]], buf_v.at[0], sem)]
        cps[0].start()
        for c in range(nch):
            cps[c].wait()
            if c + 1 < nch:
                cp = pltpu.make_async_copy(
                    table_hbm.at[idx_v.at[c + 1]], buf_v.at[(c + 1) % 2], sem)
                cp.start()
                cps.append(cp)
            pltpu.sync_copy(buf_v.at[c % 2],
                            out_hbm.at[pl.ds(base + c * chunk, chunk)])

    return k(table, idx3)


def _sc_row_scatter(values, idx, chunk):
    B, D = values.shape
    b_per_w = B // _NW
    nch = b_per_w // chunk
    idx3 = idx.reshape(_NW, nch, chunk)
    mesh = plsc.VectorSubcoreMesh(core_axis_name="c", subcore_axis_name="s")

    @functools.partial(
        pl.kernel,
        mesh=mesh,
        out_type=jax.ShapeDtypeStruct((B, D), values.dtype),
        scratch_types=[
            pltpu.VMEM((nch, chunk), jnp.int32),
            pltpu.VMEM((2, chunk, D), values.dtype),
            pltpu.SemaphoreType.DMA,
            pltpu.SemaphoreType.DMA,
        ],
    )
    def k(val_hbm, idx_hbm, out_hbm, idx_v, buf_v, sem_r, sem_w):
        wid = lax.axis_index("s") * 2 + lax.axis_index("c")
        base = wid * b_per_w
        pltpu.sync_copy(idx_hbm.at[wid], idx_v)
        rds = [pltpu.make_async_copy(
            val_hbm.at[pl.ds(base, chunk)], buf_v.at[0], sem_r)]
        rds[0].start()
        wrs = []
        for c in range(nch):
            rds[c].wait()
            wr = pltpu.make_async_copy(
                buf_v.at[c % 2], out_hbm.at[idx_v.at[c]], sem_w)
            wr.start()
            wrs.append(wr)
            if c + 1 < nch:
                if c - 1 >= 0:
                    wrs[c - 1].wait()
                rd = pltpu.make_async_copy(
                    val_hbm.at[pl.ds(base + (c + 1) * chunk, chunk)],
                    buf_v.at[(c + 1) % 2], sem_r)
                rd.start()
                rds.append(rd)
        if nch >= 2:
            wrs[nch - 2].wait()
        wrs[nch - 1].wait()

    return k(values, idx3)

_T = 512
_SR = 64
_SC = 128


def _route_body(sel_ref, inv_ref, t_ref, e_ref, lo_ref, hi_ref, init_ref,
                *, E, T, num_tiles):
    R, C = _SR, _SC
    sel2 = sel_ref[...]
    li = jax.lax.broadcasted_iota(jnp.int32, (C, C), 0)
    ci = jax.lax.broadcasted_iota(jnp.int32, (C, C), 1)
    U = (li <= ci).astype(jnp.float32)
    lr = jax.lax.broadcasted_iota(jnp.int32, (R, R), 0)
    cr = jax.lax.broadcasted_iota(jnp.int32, (R, R), 1)
    Ls = (lr > cr).astype(jnp.float32)
    ones_c = jnp.ones((C, 1), jnp.float32)

    inv2 = jnp.zeros((R, C), jnp.int32)
    offs = [jnp.int32(0)]
    for e in range(E):
        sel_is_e = sel2 == e
        m = sel_is_e.astype(jnp.float32)
        pref_in = jnp.dot(m, U, preferred_element_type=jnp.float32)
        rowtot = jnp.dot(m, ones_c, preferred_element_type=jnp.float32)
        rowpre = jnp.dot(Ls, rowtot, preferred_element_type=jnp.float32)
        rank = (pref_in - m + rowpre).astype(jnp.int32)
        inv2 = inv2 + jnp.where(sel_is_e, offs[e] + rank, 0)
        offs.append(offs[e] + jnp.sum(m).astype(jnp.int32))
    inv_ref[...] = inv2

    lane = jax.lax.broadcasted_iota(jnp.int32, (1, C), 1)
    t_v = jnp.full((1, C), num_tiles - 1, jnp.int32)
    e_v = jnp.zeros((1, C), jnp.int32)
    lo_v = jnp.zeros((1, C), jnp.int32)
    hi_v = jnp.zeros((1, C), jnp.int32)
    sstart = jnp.int32(0)
    for e in range(E):
        cnt = offs[e + 1] - offs[e]
        ft = offs[e] // T
        lt = (offs[e + 1] - 1) // T
        nt = jnp.where(cnt > 0, lt - ft + 1, 0)
        mask = (lane >= sstart) & (lane < sstart + nt)
        tt = ft + (lane - sstart)
        t_v = jnp.where(mask, tt, t_v)
        e_v = jnp.where(mask, e, e_v)
        lo_v = jnp.where(mask, jnp.maximum(offs[e], tt * T), lo_v)
        hi_v = jnp.where(mask, jnp.minimum(offs[e + 1], (tt + 1) * T), hi_v)
        sstart = sstart + nt
    tshift = pltpu.roll(t_v, 1, axis=1)
    init_v = ((t_v != tshift) | (lane == 0)).astype(jnp.int32)

    t_ref[...] = jnp.broadcast_to(t_v, (8, C))
    e_ref[...] = jnp.broadcast_to(e_v, (8, C))
    lo_ref[...] = jnp.broadcast_to(lo_v, (8, C))
    hi_ref[...] = jnp.broadcast_to(hi_v, (8, C))
    init_ref[...] = jnp.broadcast_to(init_v, (8, C))


def _gemm_body(t_ref, e_ref, lo_ref, hi_ref, init_ref, xs_ref, w_ref, out_ref):
    s = pl.program_id(0)
    t = t_ref[0, s]
    lo = lo_ref[0, s]
    hi = hi_ref[0, s]
    row = t * _T + jax.lax.broadcasted_iota(jnp.int32, (_T, 1), 0)
    mask = (row >= lo) & (row < hi)
    acc = jnp.dot(xs_ref[...].astype(jnp.bfloat16),
                  w_ref[0].astype(jnp.bfloat16),
                  preferred_element_type=jnp.float32)
    contrib = jnp.where(mask, acc, 0.0)

    @pl.when(init_ref[0, s] != 0)
    def _init():
        out_ref[...] = contrib

    @pl.when(init_ref[0, s] == 0)
    def _accum():
        out_ref[...] += contrib


def kernel(x, sel, w):
    M, K = x.shape
    E, _, N = w.shape
    T = _T
    num_tiles = M // T
    S = num_tiles + E

    i32_8x = jax.ShapeDtypeStruct((8, _SC), jnp.int32)
    inv2, t8, e8, lo8, hi8, init8 = pl.pallas_call(
        functools.partial(_route_body, E=E, T=T, num_tiles=num_tiles),
        out_shape=[jax.ShapeDtypeStruct((_SR, _SC), jnp.int32),
                   i32_8x, i32_8x, i32_8x, i32_8x, i32_8x],
    )(sel.reshape(_SR, _SC))
    inv = inv2.reshape(M)

    xs = _sc_row_scatter(x, inv, chunk=32)

    grid_spec = pltpu.PrefetchScalarGridSpec(
        num_scalar_prefetch=5,
        grid=(S,),
        in_specs=[
            pl.BlockSpec((T, K), lambda s, t, e, lo, hi, ini: (t[0, s], 0)),
            pl.BlockSpec((1, K, N), lambda s, t, e, lo, hi, ini: (e[0, s], 0, 0)),
        ],
        out_specs=pl.BlockSpec((T, N), lambda s, t, e, lo, hi, ini: (t[0, s], 0)),
    )
    ys = pl.pallas_call(
        _gemm_body,
        grid_spec=grid_spec,
        out_shape=jax.ShapeDtypeStruct((M, N), jnp.float32),
    )(t8, e8, lo8, hi8, init8, xs, w)

    return _sc_row_gather(ys, inv, chunk=64)


# speedup vs baseline: 1.0225x; 1.0225x over previous
import functools

import jax
from jax import lax
import jax.numpy as jnp
from jax.experimental import pallas as pl
from jax.experimental.pallas import tpu as pltpu
from jax.experimental.pallas import tpu_sc as plsc

_NW = 32


def _sc_row_gather(table, idx, chunk):
    V, D = table.shape
    B = idx.shape[0]
    b_per_w = B // _NW
    nch = b_per_w // chunk
    idx3 = idx.reshape(_NW, nch, chunk)
    mesh = plsc.VectorSubcoreMesh(core_axis_name="c", subcore_axis_name="s")

    @functools.partial(
        pl.kernel,
        mesh=mesh,
        out_type=jax.ShapeDtypeStruct((B, D), table.dtype),
        scratch_types=[
            pltpu.VMEM((nch, chunk), jnp.int32),
            pltpu.VMEM((2, chunk, D), table.dtype),
            pltpu.SemaphoreType.DMA,
        ],
    )
    def k(table_hbm, idx_hbm, out_hbm, idx_v, buf_v, sem):
        wid = lax.axis_index("s") * 2 + lax.axis_index("c")
        base = wid * b_per_w
        pltpu.sync_copy(idx_hbm.at[wid], idx_v)
        cps = [pltpu.make_async_copy(table_hbm.at[idx_v.at[0]], buf_v.at[0], sem)]
        cps[0].start()
        for c in range(nch):
            cps[c].wait()
            if c + 1 < nch:
                cp = pltpu.make_async_copy(
                    table_hbm.at[idx_v.at[c + 1]], buf_v.at[(c + 1) % 2], sem)
                cp.start()
                cps.append(cp)
            pltpu.sync_copy(buf_v.at[c % 2],
                            out_hbm.at[pl.ds(base + c * chunk, chunk)])

    return k(table, idx3)


def _sc_row_scatter(values, idx, chunk):
    B, D = values.shape
    b_per_w = B // _NW
    nch = b_per_w // chunk
    idx3 = idx.reshape(_NW, nch, chunk)
    mesh = plsc.VectorSubcoreMesh(core_axis_name="c", subcore_axis_name="s")

    @functools.partial(
        pl.kernel,
        mesh=mesh,
        out_type=jax.ShapeDtypeStruct((B, D), values.dtype),
        scratch_types=[
            pltpu.VMEM((nch, chunk), jnp.int32),
            pltpu.VMEM((2, chunk, D), values.dtype),
            pltpu.SemaphoreType.DMA,
            pltpu.SemaphoreType.DMA,
        ],
    )
    def k(val_hbm, idx_hbm, out_hbm, idx_v, buf_v, sem_r, sem_w):
        wid = lax.axis_index("s") * 2 + lax.axis_index("c")
        base = wid * b_per_w
        pltpu.sync_copy(idx_hbm.at[wid], idx_v)
        rds = [pltpu.make_async_copy(
            val_hbm.at[pl.ds(base, chunk)], buf_v.at[0], sem_r)]
        rds[0].start()
        wrs = []
        for c in range(nch):
            rds[c].wait()
            wr = pltpu.make_async_copy(
                buf_v.at[c % 2], out_hbm.at[idx_v.at[c]], sem_w)
            wr.start()
            wrs.append(wr)
            if c + 1 < nch:
                if c - 1 >= 0:
                    wrs[c - 1].wait()
                rd = pltpu.make_async_copy(
                    val_hbm.at[pl.ds(base + (c + 1) * chunk, chunk)],
                    buf_v.at[(c + 1) % 2], sem_r)
                rd.start()
                rds.append(rd)
        if nch >= 2:
            wrs[nch - 2].wait()
        wrs[nch - 1].wait()

    return k(values, idx3)

_T = 512
_SR = 64
_SC = 128


def _route_body(sel_ref, inv_ref, t_ref, e_ref, lo_ref, hi_ref, init_ref,
                *, E, T, num_tiles):
    R, C = _SR, _SC
    sel2 = sel_ref[...]
    li = jax.lax.broadcasted_iota(jnp.int32, (C, C), 0)
    ci = jax.lax.broadcasted_iota(jnp.int32, (C, C), 1)
    U = (li <= ci).astype(jnp.float32)
    lr = jax.lax.broadcasted_iota(jnp.int32, (R, R), 0)
    cr = jax.lax.broadcasted_iota(jnp.int32, (R, R), 1)
    Ls = (lr > cr).astype(jnp.float32)
    ones_c = jnp.ones((C, 1), jnp.float32)

    inv2 = jnp.zeros((R, C), jnp.int32)
    offs = [jnp.int32(0)]
    for e in range(E):
        sel_is_e = sel2 == e
        m = sel_is_e.astype(jnp.float32)
        pref_in = jnp.dot(m, U, preferred_element_type=jnp.float32)
        rowtot = jnp.dot(m, ones_c, preferred_element_type=jnp.float32)
        rowpre = jnp.dot(Ls, rowtot, preferred_element_type=jnp.float32)
        rank = (pref_in - m + rowpre).astype(jnp.int32)
        inv2 = inv2 + jnp.where(sel_is_e, offs[e] + rank, 0)
        offs.append(offs[e] + jnp.sum(m).astype(jnp.int32))
    inv_ref[...] = inv2

    lane = jax.lax.broadcasted_iota(jnp.int32, (1, C), 1)
    t_v = jnp.full((1, C), num_tiles - 1, jnp.int32)
    e_v = jnp.zeros((1, C), jnp.int32)
    lo_v = jnp.zeros((1, C), jnp.int32)
    hi_v = jnp.zeros((1, C), jnp.int32)
    sstart = jnp.int32(0)
    for e in range(E):
        cnt = offs[e + 1] - offs[e]
        ft = offs[e] // T
        lt = (offs[e + 1] - 1) // T
        nt = jnp.where(cnt > 0, lt - ft + 1, 0)
        mask = (lane >= sstart) & (lane < sstart + nt)
        tt = ft + (lane - sstart)
        t_v = jnp.where(mask, tt, t_v)
        e_v = jnp.where(mask, e, e_v)
        lo_v = jnp.where(mask, jnp.maximum(offs[e], tt * T), lo_v)
        hi_v = jnp.where(mask, jnp.minimum(offs[e + 1], (tt + 1) * T), hi_v)
        sstart = sstart + nt
    tshift = pltpu.roll(t_v, 1, axis=1)
    init_v = ((t_v != tshift) | (lane == 0)).astype(jnp.int32)

    t_ref[...] = jnp.broadcast_to(t_v, (8, C))
    e_ref[...] = jnp.broadcast_to(e_v, (8, C))
    lo_ref[...] = jnp.broadcast_to(lo_v, (8, C))
    hi_ref[...] = jnp.broadcast_to(hi_v, (8, C))
    init_ref[...] = jnp.broadcast_to(init_v, (8, C))


def _gemm_body(t_ref, e_ref, lo_ref, hi_ref, init_ref, xs_ref, w_ref, out_ref):
    s = pl.program_id(0)
    t = t_ref[0, s]
    lo = lo_ref[0, s]
    hi = hi_ref[0, s]
    @pl.when(hi > lo)
    def _work():
        row = t * _T + jax.lax.broadcasted_iota(jnp.int32, (_T, 1), 0)
        mask = (row >= lo) & (row < hi)
        acc = jnp.dot(xs_ref[...], w_ref[0], preferred_element_type=jnp.float32)
        contrib = jnp.where(mask, acc, 0.0)

        @pl.when(init_ref[0, s] != 0)
        def _init():
            out_ref[...] = contrib

        @pl.when(init_ref[0, s] == 0)
        def _accum():
            out_ref[...] += contrib


def kernel(x, sel, w):
    M, K = x.shape
    E, _, N = w.shape
    T = _T
    num_tiles = M // T
    S = num_tiles + E

    i32_8x = jax.ShapeDtypeStruct((8, _SC), jnp.int32)
    inv2, t8, e8, lo8, hi8, init8 = pl.pallas_call(
        functools.partial(_route_body, E=E, T=T, num_tiles=num_tiles),
        out_shape=[jax.ShapeDtypeStruct((_SR, _SC), jnp.int32),
                   i32_8x, i32_8x, i32_8x, i32_8x, i32_8x],
    )(sel.reshape(_SR, _SC))
    inv = inv2.reshape(M)

    xs = _sc_row_scatter(x, inv, chunk=32)

    grid_spec = pltpu.PrefetchScalarGridSpec(
        num_scalar_prefetch=5,
        grid=(S,),
        in_specs=[
            pl.BlockSpec((T, K), lambda s, t, e, lo, hi, ini: (t[0, s], 0)),
            pl.BlockSpec((1, K, N), lambda s, t, e, lo, hi, ini: (e[0, s], 0, 0)),
        ],
        out_specs=pl.BlockSpec((T, N), lambda s, t, e, lo, hi, ini: (t[0, s], 0)),
    )
    ys = pl.pallas_call(
        _gemm_body,
        grid_spec=grid_spec,
        out_shape=jax.ShapeDtypeStruct((M, N), jnp.float32),
    )(t8, e8, lo8, hi8, init8, xs, w)

    return _sc_row_gather(ys, inv, chunk=128)


# speedup vs baseline: 1.0683x; 1.0448x over previous
import functools

import jax
from jax import lax
import jax.numpy as jnp
from jax.experimental import pallas as pl
from jax.experimental.pallas import tpu as pltpu
from jax.experimental.pallas import tpu_sc as plsc

_NW = 32


def _sc_row_gather(table, idx, chunk):
    V, D = table.shape
    B = idx.shape[0]
    b_per_w = B // _NW
    nch = b_per_w // chunk
    idx3 = idx.reshape(_NW, nch, chunk)
    mesh = plsc.VectorSubcoreMesh(core_axis_name="c", subcore_axis_name="s")

    @functools.partial(
        pl.kernel,
        mesh=mesh,
        out_type=jax.ShapeDtypeStruct((B, D), table.dtype),
        scratch_types=[
            pltpu.VMEM((nch, chunk), jnp.int32),
            pltpu.VMEM((2, chunk, D), table.dtype),
            pltpu.SemaphoreType.DMA,
        ],
    )
    def k(table_hbm, idx_hbm, out_hbm, idx_v, buf_v, sem):
        wid = lax.axis_index("s") * 2 + lax.axis_index("c")
        base = wid * b_per_w
        pltpu.sync_copy(idx_hbm.at[wid], idx_v)
        cps = [pltpu.make_async_copy(table_hbm.at[idx_v.at[0]], buf_v.at[0], sem)]
        cps[0].start()
        for c in range(nch):
            cps[c].wait()
            if c + 1 < nch:
                cp = pltpu.make_async_copy(
                    table_hbm.at[idx_v.at[c + 1]], buf_v.at[(c + 1) % 2], sem)
                cp.start()
                cps.append(cp)
            pltpu.sync_copy(buf_v.at[c % 2],
                            out_hbm.at[pl.ds(base + c * chunk, chunk)])

    return k(table, idx3)


def _sc_row_scatter(values, idx, chunk):
    B, D = values.shape
    b_per_w = B // _NW
    nch = b_per_w // chunk
    idx3 = idx.reshape(_NW, nch, chunk)
    mesh = plsc.VectorSubcoreMesh(core_axis_name="c", subcore_axis_name="s")
    nbuf = 3

    @functools.partial(
        pl.kernel,
        mesh=mesh,
        out_type=jax.ShapeDtypeStruct((B, D), values.dtype),
        scratch_types=[
            pltpu.VMEM((nch, chunk), jnp.int32),
            pltpu.VMEM((nbuf, chunk, D), values.dtype),
            pltpu.SemaphoreType.DMA,
            pltpu.SemaphoreType.DMA,
        ],
    )
    def k(val_hbm, idx_hbm, out_hbm, idx_v, buf_v, sem_r, sem_w):
        wid = lax.axis_index("s") * 2 + lax.axis_index("c")
        base = wid * b_per_w
        rds = []
        for c in range(min(nbuf, nch)):
            rd = pltpu.make_async_copy(
                val_hbm.at[pl.ds(base + c * chunk, chunk)],
                buf_v.at[c % nbuf], sem_r)
            rd.start()
            rds.append(rd)
        pltpu.sync_copy(idx_hbm.at[wid], idx_v)
        wrs = []
        for c in range(nch):
            rds[c].wait()
            wr = pltpu.make_async_copy(
                buf_v.at[c % nbuf], out_hbm.at[idx_v.at[c]], sem_w)
            wr.start()
            wrs.append(wr)
            if c + nbuf < nch:
                wrs[c].wait()
                rd = pltpu.make_async_copy(
                    val_hbm.at[pl.ds(base + (c + nbuf) * chunk, chunk)],
                    buf_v.at[c % nbuf], sem_r)
                rd.start()
                rds.append(rd)
        for c in range(max(0, nch - nbuf), nch):
            wrs[c].wait()

    return k(values, idx3)

_T = 512
_SR = 64
_SC = 128


def _route_body(sel_ref, inv_ref, t_ref, e_ref, lo_ref, hi_ref, init_ref,
                *, E, T, num_tiles):
    R, C = _SR, _SC
    sel2 = sel_ref[...]
    li = jax.lax.broadcasted_iota(jnp.int32, (C, C), 0)
    ci = jax.lax.broadcasted_iota(jnp.int32, (C, C), 1)
    U = (li <= ci).astype(jnp.float32)
    lr = jax.lax.broadcasted_iota(jnp.int32, (R, R), 0)
    cr = jax.lax.broadcasted_iota(jnp.int32, (R, R), 1)
    Ls = (lr > cr).astype(jnp.float32)
    ones_c = jnp.ones((C, 1), jnp.float32)

    inv2 = jnp.zeros((R, C), jnp.int32)
    offs = [jnp.int32(0)]
    for e in range(E):
        sel_is_e = sel2 == e
        m = sel_is_e.astype(jnp.float32)
        pref_in = jnp.dot(m, U, preferred_element_type=jnp.float32)
        rowtot = jnp.dot(m, ones_c, preferred_element_type=jnp.float32)
        rowpre = jnp.dot(Ls, rowtot, preferred_element_type=jnp.float32)
        rank = (pref_in - m + rowpre).astype(jnp.int32)
        inv2 = inv2 + jnp.where(sel_is_e, offs[e] + rank, 0)
        offs.append(offs[e] + jnp.sum(m).astype(jnp.int32))
    inv_ref[...] = inv2

    lane = jax.lax.broadcasted_iota(jnp.int32, (1, C), 1)
    t_v = jnp.full((1, C), num_tiles - 1, jnp.int32)
    e_v = jnp.zeros((1, C), jnp.int32)
    lo_v = jnp.zeros((1, C), jnp.int32)
    hi_v = jnp.zeros((1, C), jnp.int32)
    sstart = jnp.int32(0)
    for e in range(E):
        cnt = offs[e + 1] - offs[e]
        ft = offs[e] // T
        lt = (offs[e + 1] - 1) // T
        nt = jnp.where(cnt > 0, lt - ft + 1, 0)
        mask = (lane >= sstart) & (lane < sstart + nt)
        tt = ft + (lane - sstart)
        t_v = jnp.where(mask, tt, t_v)
        e_v = jnp.where(mask, e, e_v)
        lo_v = jnp.where(mask, jnp.maximum(offs[e], tt * T), lo_v)
        hi_v = jnp.where(mask, jnp.minimum(offs[e + 1], (tt + 1) * T), hi_v)
        sstart = sstart + nt
    tshift = pltpu.roll(t_v, 1, axis=1)
    init_v = ((t_v != tshift) | (lane == 0)).astype(jnp.int32)

    t_ref[...] = jnp.broadcast_to(t_v, (8, C))
    e_ref[...] = jnp.broadcast_to(e_v, (8, C))
    lo_ref[...] = jnp.broadcast_to(lo_v, (8, C))
    hi_ref[...] = jnp.broadcast_to(hi_v, (8, C))
    init_ref[...] = jnp.broadcast_to(init_v, (8, C))


def _gemm_body(t_ref, e_ref, lo_ref, hi_ref, init_ref, xs_ref, w_ref, out_ref):
    s = pl.program_id(0)
    t = t_ref[0, s]
    lo = lo_ref[0, s]
    hi = hi_ref[0, s]
    @pl.when(hi > lo)
    def _work():
        row = t * _T + jax.lax.broadcasted_iota(jnp.int32, (_T, 1), 0)
        mask = (row >= lo) & (row < hi)
        acc = jnp.dot(xs_ref[...], w_ref[0], preferred_element_type=jnp.float32)
        contrib = jnp.where(mask, acc, 0.0)

        @pl.when(init_ref[0, s] != 0)
        def _init():
            out_ref[...] = contrib

        @pl.when(init_ref[0, s] == 0)
        def _accum():
            out_ref[...] += contrib


def kernel(x, sel, w):
    M, K = x.shape
    E, _, N = w.shape
    T = _T
    num_tiles = M // T
    S = num_tiles + E

    i32_8x = jax.ShapeDtypeStruct((8, _SC), jnp.int32)
    inv2, t8, e8, lo8, hi8, init8 = pl.pallas_call(
        functools.partial(_route_body, E=E, T=T, num_tiles=num_tiles),
        out_shape=[jax.ShapeDtypeStruct((_SR, _SC), jnp.int32),
                   i32_8x, i32_8x, i32_8x, i32_8x, i32_8x],
    )(sel.reshape(_SR, _SC))
    inv = inv2.reshape(M)

    xs = _sc_row_scatter(x, inv, chunk=32)

    grid_spec = pltpu.PrefetchScalarGridSpec(
        num_scalar_prefetch=5,
        grid=(S,),
        in_specs=[
            pl.BlockSpec((T, K), lambda s, t, e, lo, hi, ini: (t[0, s], 0)),
            pl.BlockSpec((1, K, N), lambda s, t, e, lo, hi, ini: (e[0, s], 0, 0)),
        ],
        out_specs=pl.BlockSpec((T, N), lambda s, t, e, lo, hi, ini: (t[0, s], 0)),
    )
    ys = pl.pallas_call(
        _gemm_body,
        grid_spec=grid_spec,
        out_shape=jax.ShapeDtypeStruct((M, N), jnp.float32),
    )(t8, e8, lo8, hi8, init8, xs, w)

    return _sc_row_gather(ys, inv, chunk=128)


# speedup vs baseline: 1.0783x; 1.0093x over previous
import functools

import jax
from jax import lax
import jax.numpy as jnp
from jax.experimental import pallas as pl
from jax.experimental.pallas import tpu as pltpu
from jax.experimental.pallas import tpu_sc as plsc

_NW = 32
_NBUF = 3


def _sc_row_gather(table, idx, chunk):
    V, D = table.shape
    B = idx.shape[0]
    b_per_w = B // _NW
    nch = b_per_w // chunk
    idx3 = idx.reshape(_NW, nch, chunk)
    mesh = plsc.VectorSubcoreMesh(core_axis_name="c", subcore_axis_name="s")

    @functools.partial(
        pl.kernel,
        mesh=mesh,
        out_type=jax.ShapeDtypeStruct((B, D), table.dtype),
        scratch_types=[
            pltpu.VMEM((nch, chunk), jnp.int32),
            pltpu.VMEM((_NBUF, chunk, D), table.dtype),
            pltpu.SemaphoreType.DMA,
            pltpu.SemaphoreType.DMA,
        ],
    )
    def k(table_hbm, idx_hbm, out_hbm, idx_v, buf_v, sem, sem_w):
        wid = lax.axis_index("s") * 2 + lax.axis_index("c")
        base = wid * b_per_w
        pltpu.sync_copy(idx_hbm.at[wid], idx_v)
        cps = []
        for c in range(min(_NBUF, nch)):
            cp = pltpu.make_async_copy(
                table_hbm.at[idx_v.at[c]], buf_v.at[c % _NBUF], sem)
            cp.start()
            cps.append(cp)
        wrs = []
        for c in range(nch):
            cps[c].wait()
            wr = pltpu.make_async_copy(
                buf_v.at[c % _NBUF],
                out_hbm.at[pl.ds(base + c * chunk, chunk)], sem_w)
            wr.start()
            wrs.append(wr)
            if c + _NBUF < nch:
                wrs[c].wait()
                cp = pltpu.make_async_copy(
                    table_hbm.at[idx_v.at[c + _NBUF]], buf_v.at[c % _NBUF], sem)
                cp.start()
                cps.append(cp)
        for c in range(max(0, nch - _NBUF), nch):
            wrs[c].wait()

    return k(table, idx3)


def _sc_row_scatter(values, idx, chunk):
    B, D = values.shape
    b_per_w = B // _NW
    nch = b_per_w // chunk
    idx3 = idx.reshape(_NW, nch, chunk)
    mesh = plsc.VectorSubcoreMesh(core_axis_name="c", subcore_axis_name="s")
    nbuf = 3

    @functools.partial(
        pl.kernel,
        mesh=mesh,
        out_type=jax.ShapeDtypeStruct((B, D), values.dtype),
        scratch_types=[
            pltpu.VMEM((nch, chunk), jnp.int32),
            pltpu.VMEM((nbuf, chunk, D), values.dtype),
            pltpu.SemaphoreType.DMA,
            pltpu.SemaphoreType.DMA,
        ],
    )
    def k(val_hbm, idx_hbm, out_hbm, idx_v, buf_v, sem_r, sem_w):
        wid = lax.axis_index("s") * 2 + lax.axis_index("c")
        base = wid * b_per_w
        rds = []
        for c in range(min(nbuf, nch)):
            rd = pltpu.make_async_copy(
                val_hbm.at[pl.ds(base + c * chunk, chunk)],
                buf_v.at[c % nbuf], sem_r)
            rd.start()
            rds.append(rd)
        pltpu.sync_copy(idx_hbm.at[wid], idx_v)
        wrs = []
        for c in range(nch):
            rds[c].wait()
            wr = pltpu.make_async_copy(
                buf_v.at[c % nbuf], out_hbm.at[idx_v.at[c]], sem_w)
            wr.start()
            wrs.append(wr)
            if c + nbuf < nch:
                wrs[c].wait()
                rd = pltpu.make_async_copy(
                    val_hbm.at[pl.ds(base + (c + nbuf) * chunk, chunk)],
                    buf_v.at[c % nbuf], sem_r)
                rd.start()
                rds.append(rd)
        for c in range(max(0, nch - nbuf), nch):
            wrs[c].wait()

    return k(values, idx3)

_T = 512
_SR = 64
_SC = 128


def _route_body(sel_ref, inv_ref, t_ref, e_ref, lo_ref, hi_ref, init_ref,
                *, E, T, num_tiles):
    R, C = _SR, _SC
    sel2 = sel_ref[...]
    li = jax.lax.broadcasted_iota(jnp.int32, (C, C), 0)
    ci = jax.lax.broadcasted_iota(jnp.int32, (C, C), 1)
    U = (li <= ci).astype(jnp.float32)
    lr = jax.lax.broadcasted_iota(jnp.int32, (R, R), 0)
    cr = jax.lax.broadcasted_iota(jnp.int32, (R, R), 1)
    Ls = (lr > cr).astype(jnp.float32)
    ones_c = jnp.ones((C, 1), jnp.float32)

    inv2 = jnp.zeros((R, C), jnp.int32)
    offs = [jnp.int32(0)]
    for e in range(E):
        sel_is_e = sel2 == e
        m = sel_is_e.astype(jnp.float32)
        pref_in = jnp.dot(m, U, preferred_element_type=jnp.float32)
        rowtot = jnp.dot(m, ones_c, preferred_element_type=jnp.float32)
        rowpre = jnp.dot(Ls, rowtot, preferred_element_type=jnp.float32)
        rank = (pref_in - m + rowpre).astype(jnp.int32)
        inv2 = inv2 + jnp.where(sel_is_e, offs[e] + rank, 0)
        offs.append(offs[e] + jnp.sum(m).astype(jnp.int32))
    inv_ref[...] = inv2

    lane = jax.lax.broadcasted_iota(jnp.int32, (1, C), 1)
    t_v = jnp.full((1, C), num_tiles - 1, jnp.int32)
    e_v = jnp.zeros((1, C), jnp.int32)
    lo_v = jnp.zeros((1, C), jnp.int32)
    hi_v = jnp.zeros((1, C), jnp.int32)
    sstart = jnp.int32(0)
    for e in range(E):
        cnt = offs[e + 1] - offs[e]
        ft = offs[e] // T
        lt = (offs[e + 1] - 1) // T
        nt = jnp.where(cnt > 0, lt - ft + 1, 0)
        mask = (lane >= sstart) & (lane < sstart + nt)
        tt = ft + (lane - sstart)
        t_v = jnp.where(mask, tt, t_v)
        e_v = jnp.where(mask, e, e_v)
        lo_v = jnp.where(mask, jnp.maximum(offs[e], tt * T), lo_v)
        hi_v = jnp.where(mask, jnp.minimum(offs[e + 1], (tt + 1) * T), hi_v)
        sstart = sstart + nt
    tshift = pltpu.roll(t_v, 1, axis=1)
    init_v = ((t_v != tshift) | (lane == 0)).astype(jnp.int32)

    t_ref[...] = jnp.broadcast_to(t_v, (8, C))
    e_ref[...] = jnp.broadcast_to(e_v, (8, C))
    lo_ref[...] = jnp.broadcast_to(lo_v, (8, C))
    hi_ref[...] = jnp.broadcast_to(hi_v, (8, C))
    init_ref[...] = jnp.broadcast_to(init_v, (8, C))


def _gemm_body(t_ref, e_ref, lo_ref, hi_ref, init_ref, xs_ref, w_ref, out_ref):
    s = pl.program_id(0)
    t = t_ref[0, s]
    lo = lo_ref[0, s]
    hi = hi_ref[0, s]
    @pl.when(hi > lo)
    def _work():
        row = t * _T + jax.lax.broadcasted_iota(jnp.int32, (_T, 1), 0)
        mask = (row >= lo) & (row < hi)
        acc = jnp.dot(xs_ref[...], w_ref[0], preferred_element_type=jnp.float32)
        contrib = jnp.where(mask, acc, 0.0)

        @pl.when(init_ref[0, s] != 0)
        def _init():
            out_ref[...] = contrib

        @pl.when(init_ref[0, s] == 0)
        def _accum():
            out_ref[...] += contrib


def kernel(x, sel, w):
    M, K = x.shape
    E, _, N = w.shape
    T = _T
    num_tiles = M // T
    S = num_tiles + E

    i32_8x = jax.ShapeDtypeStruct((8, _SC), jnp.int32)
    inv2, t8, e8, lo8, hi8, init8 = pl.pallas_call(
        functools.partial(_route_body, E=E, T=T, num_tiles=num_tiles),
        out_shape=[jax.ShapeDtypeStruct((_SR, _SC), jnp.int32),
                   i32_8x, i32_8x, i32_8x, i32_8x, i32_8x],
    )(sel.reshape(_SR, _SC))
    inv = inv2.reshape(M)

    xs = _sc_row_scatter(x, inv, chunk=32)

    grid_spec = pltpu.PrefetchScalarGridSpec(
        num_scalar_prefetch=5,
        grid=(S,),
        in_specs=[
            pl.BlockSpec((T, K), lambda s, t, e, lo, hi, ini: (t[0, s], 0)),
            pl.BlockSpec((1, K, N), lambda s, t, e, lo, hi, ini: (e[0, s], 0, 0)),
        ],
        out_specs=pl.BlockSpec((T, N), lambda s, t, e, lo, hi, ini: (t[0, s], 0)),
    )
    ys = pl.pallas_call(
        _gemm_body,
        grid_spec=grid_spec,
        out_shape=jax.ShapeDtypeStruct((M, N), jnp.float32),
    )(t8, e8, lo8, hi8, init8, xs, w)

    return _sc_row_gather(ys, inv, chunk=64)
